# baseline (speedup 1.0000x reference)
import functools

import jax
import jax.numpy as jnp
from jax import lax
from jax.experimental import pallas as pl
from jax.experimental.pallas import tpu as pltpu
from jax.experimental.pallas import tpu_sc as plsc

NC = 2
NS = 16
L = 16
CHUNK = 128
W = 128
NWIN = 7813
NSLOT = 4
GROW = 128


def _extract_body(batch, ut_hbm, it_hbm, usr_hbm, itm_hbm, ug_hbm, ig_hbm,
                  idxv, hitsv, hposv, wslots, colbuf, posbuf,
                  sem0, sem1, sem2, sem3, ssem):
  sems = (sem0, sem1, sem2, sem3)
  wid = lax.axis_index("s") * NC + lax.axis_index("c")
  nw = NC * NS
  wper = NWIN // nw
  wext = NWIN - wper * nw
  wlo = wid * wper + jnp.minimum(wid, wext)
  nwin = wper + (wid < wext).astype(jnp.int32)
  n_macro = (wper + 1 + NSLOT - 1) // NSLOT

  lane = lax.iota(jnp.int32, L)
  nvreg = batch // L

  for side_hbm, tab_hbm, g_hbm in ((usr_hbm, ut_hbm, ug_hbm),
                                   (itm_hbm, it_hbm, ig_hbm)):
    pltpu.sync_copy(side_hbm, idxv.at[pl.ds(0, batch)])

    def compact(t, cnt):
      v = idxv[pl.ds(t * L, L)]
      w = jax.lax.shift_right_logical(v, 7)
      m = (w >= wlo) & (w < wlo + nwin)
      plsc.store_compressed(hitsv.at[pl.ds(cnt, L)], v, mask=m)
      plsc.store_compressed(hposv.at[pl.ds(cnt, L)], t * L + lane, mask=m)
      return cnt + plsc.all_reduce_population_count(m)[0]

    cnt = lax.fori_loop(0, nvreg, compact, jnp.int32(0))
    hitsv[pl.ds(cnt, L)] = jnp.full((L,), -1, jnp.int32)
    nhv = (cnt + L - 1) // L

    for t in range(GROW // L):
      posbuf[0, pl.ds(t * L, L)] = jnp.full((L,), batch, jnp.int32)

    def fire(k, slot):
      wk = wlo + jnp.minimum(k, nwin - 1)
      ua = pl.multiple_of(wk * W, W)
      pltpu.async_copy(tab_hbm.at[:, pl.ds(ua, W)], wslots.at[slot],
                       sems[slot])

    def drain(slot):
      pltpu.make_async_copy(tab_hbm.at[:, pl.ds(0, W)], wslots.at[slot],
                            sems[slot]).wait()

    def flush(slotc):
      pltpu.sync_copy(colbuf, g_hbm.at[posbuf.at[0]])
      for t in range(GROW // L):
        posbuf[0, pl.ds(t * L, L)] = jnp.full((L,), batch, jnp.int32)

    for s in range(NSLOT - 1):
      fire(s, s)

    def macro(mi, slotc):
      for s in range(NSLOT):
        k = mi * NSLOT + s
        fire(k + NSLOT - 1, (s + NSLOT - 1) % NSLOT)
        drain(s)
        wk = wlo + jnp.minimum(k, nwin - 1)

        def scan_vreg(t, slotc):
          hv = hitsv[pl.ds(t * L, L)]
          pv = hposv[pl.ds(t * L, L)]
          m = jax.lax.shift_right_logical(hv, 7) == wk

          def has_hits(carry):
            m, _ = carry
            return jnp.any(m)

          def take_hit(carry):
            m, slotc = carry
            l = plsc.all_reduce_ffs(m)
            us = plsc.load_gather(hitsv, [t * L + l])
            ps = plsc.load_gather(hposv, [t * L + l])
            col = us & (W - 1)
            for c in range(4):
              uv = plsc.load_gather(
                  wslots, [jnp.full((L,), s, jnp.int32), c * L + lane, col])
              colbuf[slotc & (GROW - 1), pl.ds(c * L, L)] = uv
            plsc.store_scatter(
                posbuf, [jnp.zeros((L,), jnp.int32),
                         jnp.zeros((L,), jnp.int32) + (slotc & (GROW - 1))],
                ps)
            slotc = slotc + 1

            @pl.when(slotc & (GROW - 1) == 0)
            def _():
              flush(slotc)

            return m & (lane != l), slotc

          _, slotc = lax.while_loop(has_hits, take_hit, (m, slotc))
          return slotc

        slotc = lax.fori_loop(0, nhv, scan_vreg, slotc)
      return slotc

    slotc = lax.fori_loop(0, n_macro, macro, jnp.int32(0))
    for s in range(NSLOT - 1):
      drain(s)

    @pl.when(slotc & (GROW - 1) != 0)
    def _():
      flush(slotc)


def _dot_body(batch, b_per_w, ug_hbm, ig_hbm, ub_hbm, ib_hbm, usr_hbm,
              itm_hbm, out_hbm, usr_v, itm_v, urows, irows, ubv, ibv,
              outv, bsem, csem):
  wid = lax.axis_index("s") * NC + lax.axis_index("c")
  base = wid * b_per_w

  pltpu.sync_copy(usr_hbm.at[pl.ds(base, b_per_w)], usr_v)
  pltpu.sync_copy(itm_hbm.at[pl.ds(base, b_per_w)], itm_v)
  bias_copies = []
  for g in range(b_per_w // CHUNK):
    sl = pl.ds(g * CHUNK, CHUNK)
    bias_copies.append(
        pltpu.async_copy(ub_hbm.at[usr_v.at[sl]], ubv.at[sl], bsem))
    bias_copies.append(
        pltpu.async_copy(ib_hbm.at[itm_v.at[sl]], ibv.at[sl], bsem))
  for c in bias_copies:
    c.wait()

  lane = lax.iota(jnp.int32, L)
  dnums = lax.GatherDimensionNumbers(
      offset_dims=(), collapsed_slice_dims=(0,), start_index_map=(0,))

  def shufxor(x, k):
    return lax.gather(x, (lane ^ k)[:, None], dnums, (1,),
                      mode=lax.GatherScatterMode.PROMISE_IN_BOUNDS)

  gper = GROW // L

  def group(g, carry):
    @pl.when(g % gper == 0)
    def _():
      ch = g // gper
      rsl = pl.ds(base + ch * GROW, GROW)
      pltpu.sync_copy(ug_hbm.at[rsl], urows)
      pltpu.sync_copy(ig_hbm.at[rsl], irows)

    vecs = []
    for j in range(L):
      r = (g % gper) * L + j
      acc = urows[r, pl.ds(0, L)] * irows[r, pl.ds(0, L)]
      for c in range(1, 4):
        acc = acc + urows[r, pl.ds(c * L, L)] * irows[r, pl.ds(c * L, L)]
      vecs.append(acc)
    for k in (1, 2, 4, 8):
      nxt = []
      sel = (lane & k) == 0
      for p in range(0, len(vecs), 2):
        a, b = vecs[p], vecs[p + 1]
        nxt.append(jnp.where(sel, a + shufxor(a, k), b + shufxor(b, k)))
      vecs = nxt
    sl = pl.ds(g * L, L)
    outv[sl] = vecs[0] + ubv[sl] + ibv[sl]
    return carry

  lax.fori_loop(0, b_per_w // L, group, 0)
  pltpu.sync_copy(outv, out_hbm.at[pl.ds(base, b_per_w)])


def kernel(uEmbeds, iEmbeds, uBias, iBias, usr, itm):
  batch = usr.shape[0]
  latdim = uEmbeds.shape[1]
  nw = NC * NS
  b_per_w = batch // nw
  uT = uEmbeds.T
  iT = iEmbeds.T
  mesh = plsc.VectorSubcoreMesh(
      core_axis_name="c", subcore_axis_name="s", num_cores=NC,
      num_subcores=NS)
  params = pltpu.CompilerParams(
      use_tc_tiling_on_sc=True, needs_layout_passes=False)

  extract = pl.kernel(
      functools.partial(_extract_body, batch),
      out_type=(
          jax.ShapeDtypeStruct((batch + GROW, W), jnp.float32),
          jax.ShapeDtypeStruct((batch + GROW, W), jnp.float32),
      ),
      mesh=mesh,
      scratch_types=[
          pltpu.VMEM((batch,), jnp.int32),
          pltpu.VMEM((batch + L,), jnp.int32),
          pltpu.VMEM((batch + L,), jnp.int32),
          pltpu.VMEM((NSLOT, latdim, W), jnp.float32),
          pltpu.VMEM((GROW, W), jnp.float32),
          pltpu.VMEM((1, GROW), jnp.int32),
          pltpu.SemaphoreType.DMA,
          pltpu.SemaphoreType.DMA,
          pltpu.SemaphoreType.DMA,
          pltpu.SemaphoreType.DMA,
          pltpu.SemaphoreType.DMA,
      ],
      compiler_params=params,
  )
  uG, iG = extract(uT, iT, usr, itm)

  dot = pl.kernel(
      functools.partial(_dot_body, batch, b_per_w),
      out_type=jax.ShapeDtypeStruct((batch,), jnp.float32),
      mesh=mesh,
      scratch_types=[
          pltpu.VMEM((b_per_w,), jnp.int32),
          pltpu.VMEM((b_per_w,), jnp.int32),
          pltpu.VMEM((GROW, W), jnp.float32),
          pltpu.VMEM((GROW, W), jnp.float32),
          pltpu.VMEM((b_per_w,), jnp.float32),
          pltpu.VMEM((b_per_w,), jnp.float32),
          pltpu.VMEM((b_per_w,), jnp.float32),
          pltpu.SemaphoreType.DMA,
          pltpu.SemaphoreType.DMA,
      ],
      compiler_params=params,
  )
  return dot(uG, iG, uBias, iBias, usr, itm)


# speedup vs baseline: 1.4193x; 1.4193x over previous
import functools

import jax
import jax.numpy as jnp
from jax import lax
from jax.experimental import pallas as pl
from jax.experimental.pallas import tpu as pltpu
from jax.experimental.pallas import tpu_sc as plsc

NC = 2
NS = 16
L = 16
CHUNK = 128
W = 128
NWIN = 7813
NSLOT = 8
GROW = 128


def _extract_body(batch, ut_hbm, it_hbm, usr_hbm, itm_hbm, ug_hbm, ig_hbm,
                  idxv, hitsv, hposv, wslots, colbuf, posbuf,
                  sem0, sem1, sem2, sem3, sem4, sem5, sem6, sem7, ssem):
  sems = (sem0, sem1, sem2, sem3, sem4, sem5, sem6, sem7)
  wid = lax.axis_index("s") * NC + lax.axis_index("c")
  nw = NC * NS
  wper = NWIN // nw
  wext = NWIN - wper * nw
  wlo = wid * wper + jnp.minimum(wid, wext)
  nwin = wper + (wid < wext).astype(jnp.int32)
  MB = NSLOT // 2
  n_macro = (wper + 1 + MB - 1) // MB

  lane = lax.iota(jnp.int32, L)
  ihalf = batch // 2

  for side_hbm, tab_hbm, g_hbm in ((usr_hbm, ut_hbm, ug_hbm),
                                   (itm_hbm, it_hbm, ig_hbm)):
    def compact(t, cnt):
      v = idxv[pl.ds((t % (ihalf // L)) * L, L)]
      w = jax.lax.shift_right_logical(v, 7)
      m = (w >= wlo) & (w < wlo + nwin)
      plsc.store_compressed(hitsv.at[pl.ds(cnt, L)], v, mask=m)
      plsc.store_compressed(hposv.at[pl.ds(cnt, L)], t * L + lane, mask=m)
      return cnt + plsc.all_reduce_population_count(m)[0]

    cnt = jnp.int32(0)
    for h in range(2):
      pltpu.sync_copy(side_hbm.at[pl.ds(h * ihalf, ihalf)], idxv)
      cnt = lax.fori_loop(h * (ihalf // L), (h + 1) * (ihalf // L),
                          compact, cnt)
    hitsv[pl.ds(cnt, L)] = jnp.full((L,), -1, jnp.int32)
    nhv = (cnt + L - 1) // L

    for t in range(GROW // L):
      posbuf[0, pl.ds(t * L, L)] = jnp.full((L,), batch, jnp.int32)

    def fire(k, slot):
      wk = wlo + jnp.minimum(k, nwin - 1)
      ua = pl.multiple_of(wk * W, W)
      pltpu.async_copy(tab_hbm.at[:, pl.ds(ua, W)], wslots.at[slot],
                       sems[slot])

    def drain(slot):
      pltpu.make_async_copy(tab_hbm.at[:, pl.ds(0, W)], wslots.at[slot],
                            sems[slot]).wait()

    def flush():
      pltpu.sync_copy(colbuf, g_hbm.at[posbuf.at[0]])
      for t in range(GROW // L):
        posbuf[0, pl.ds(t * L, L)] = jnp.full((L,), batch, jnp.int32)

    for s in range(MB):
      fire(s, s)

    def macro_pair(p, slotc):
      for half in range(2):
        mi = p * 2 + half
        k4 = mi * MB
        sbase = (half * MB) % NSLOT
        for s in range(MB):
          fire(k4 + MB + s, (sbase + MB + s) % NSLOT)
        for s in range(MB):
          drain(sbase + s)

        def scan_vreg(t, slotc):
          hv = hitsv[pl.ds(t * L, L)]
          d = jax.lax.shift_right_logical(hv, 7) - wlo
          m = (d >= k4) & (d < k4 + MB) & (d < nwin)

          def has_hits(carry):
            m, _ = carry
            return jnp.any(m)

          def take_hit(carry):
            m, slotc = carry
            l = plsc.all_reduce_ffs(m)
            us = plsc.load_gather(hitsv, [t * L + l])
            ps = plsc.load_gather(hposv, [t * L + l])
            slot = (jax.lax.shift_right_logical(us, 7) - wlo) & (NSLOT - 1)
            col = us & (W - 1)
            for c in range(4):
              uv = plsc.load_gather(wslots, [slot, c * L + lane, col])
              colbuf[slotc & (GROW - 1), pl.ds(c * L, L)] = uv
            plsc.store_scatter(
                posbuf, [jnp.zeros((L,), jnp.int32),
                         jnp.zeros((L,), jnp.int32) + (slotc & (GROW - 1))],
                ps)
            slotc = slotc + 1

            @pl.when(slotc & (GROW - 1) == 0)
            def _():
              flush()

            return m & (lane != l), slotc

          _, slotc = lax.while_loop(has_hits, take_hit, (m, slotc))
          return slotc

        slotc = lax.fori_loop(0, nhv, scan_vreg, slotc)
      return slotc

    slotc = lax.fori_loop(0, n_macro // 2, macro_pair, jnp.int32(0))
    for s in range(MB):
      drain(s)

    @pl.when(slotc & (GROW - 1) != 0)
    def _():
      flush()


def _dot_body(batch, b_per_w, ug_hbm, ig_hbm, ub_hbm, ib_hbm, usr_hbm,
              itm_hbm, out_hbm, usr_v, itm_v, urows, irows, ubv, ibv,
              outv, bsem, csem):
  wid = lax.axis_index("s") * NC + lax.axis_index("c")
  base = wid * b_per_w

  pltpu.sync_copy(usr_hbm.at[pl.ds(base, b_per_w)], usr_v)
  pltpu.sync_copy(itm_hbm.at[pl.ds(base, b_per_w)], itm_v)
  bias_copies = []
  for g in range(b_per_w // CHUNK):
    sl = pl.ds(g * CHUNK, CHUNK)
    bias_copies.append(
        pltpu.async_copy(ub_hbm.at[usr_v.at[sl]], ubv.at[sl], bsem))
    bias_copies.append(
        pltpu.async_copy(ib_hbm.at[itm_v.at[sl]], ibv.at[sl], bsem))
  for c in bias_copies:
    c.wait()

  lane = lax.iota(jnp.int32, L)
  dnums = lax.GatherDimensionNumbers(
      offset_dims=(), collapsed_slice_dims=(0,), start_index_map=(0,))

  def shufxor(x, k):
    return lax.gather(x, (lane ^ k)[:, None], dnums, (1,),
                      mode=lax.GatherScatterMode.PROMISE_IN_BOUNDS)

  gper = GROW // L

  def group(g, carry):
    @pl.when(g % gper == 0)
    def _():
      ch = g // gper
      rsl = pl.ds(base + ch * GROW, GROW)
      pltpu.sync_copy(ug_hbm.at[rsl], urows)
      pltpu.sync_copy(ig_hbm.at[rsl], irows)

    vecs = []
    for j in range(L):
      r = (g % gper) * L + j
      acc = urows[r, pl.ds(0, L)] * irows[r, pl.ds(0, L)]
      for c in range(1, 4):
        acc = acc + urows[r, pl.ds(c * L, L)] * irows[r, pl.ds(c * L, L)]
      vecs.append(acc)
    for k in (1, 2, 4, 8):
      nxt = []
      sel = (lane & k) == 0
      for p in range(0, len(vecs), 2):
        a, b = vecs[p], vecs[p + 1]
        nxt.append(jnp.where(sel, a + shufxor(a, k), b + shufxor(b, k)))
      vecs = nxt
    sl = pl.ds(g * L, L)
    outv[sl] = vecs[0] + ubv[sl] + ibv[sl]
    return carry

  lax.fori_loop(0, b_per_w // L, group, 0)
  pltpu.sync_copy(outv, out_hbm.at[pl.ds(base, b_per_w)])


def kernel(uEmbeds, iEmbeds, uBias, iBias, usr, itm):
  batch = usr.shape[0]
  latdim = uEmbeds.shape[1]
  nw = NC * NS
  b_per_w = batch // nw
  uT = uEmbeds.T
  iT = iEmbeds.T
  mesh = plsc.VectorSubcoreMesh(
      core_axis_name="c", subcore_axis_name="s", num_cores=NC,
      num_subcores=NS)
  params = pltpu.CompilerParams(
      use_tc_tiling_on_sc=True, needs_layout_passes=False)

  extract = pl.kernel(
      functools.partial(_extract_body, batch),
      out_type=(
          jax.ShapeDtypeStruct((batch + GROW, W), jnp.float32),
          jax.ShapeDtypeStruct((batch + GROW, W), jnp.float32),
      ),
      mesh=mesh,
      scratch_types=[
          pltpu.VMEM((batch // 2,), jnp.int32),
          pltpu.VMEM((batch + L,), jnp.int32),
          pltpu.VMEM((batch + L,), jnp.int32),
          pltpu.VMEM((NSLOT, latdim, W), jnp.float32),
          pltpu.VMEM((GROW, W), jnp.float32),
          pltpu.VMEM((1, GROW), jnp.int32),
          pltpu.SemaphoreType.DMA,
          pltpu.SemaphoreType.DMA,
          pltpu.SemaphoreType.DMA,
          pltpu.SemaphoreType.DMA,
          pltpu.SemaphoreType.DMA,
          pltpu.SemaphoreType.DMA,
          pltpu.SemaphoreType.DMA,
          pltpu.SemaphoreType.DMA,
          pltpu.SemaphoreType.DMA,
      ],
      compiler_params=params,
  )
  uG, iG = extract(uT, iT, usr, itm)

  dot = pl.kernel(
      functools.partial(_dot_body, batch, b_per_w),
      out_type=jax.ShapeDtypeStruct((batch,), jnp.float32),
      mesh=mesh,
      scratch_types=[
          pltpu.VMEM((b_per_w,), jnp.int32),
          pltpu.VMEM((b_per_w,), jnp.int32),
          pltpu.VMEM((GROW, W), jnp.float32),
          pltpu.VMEM((GROW, W), jnp.float32),
          pltpu.VMEM((b_per_w,), jnp.float32),
          pltpu.VMEM((b_per_w,), jnp.float32),
          pltpu.VMEM((b_per_w,), jnp.float32),
          pltpu.SemaphoreType.DMA,
          pltpu.SemaphoreType.DMA,
      ],
      compiler_params=params,
  )
  return dot(uG, iG, uBias, iBias, usr, itm)


# speedup vs baseline: 1.8329x; 1.2914x over previous
import functools

import jax
import jax.numpy as jnp
from jax import lax
from jax.experimental import pallas as pl
from jax.experimental.pallas import tpu as pltpu
from jax.experimental.pallas import tpu_sc as plsc

NC = 2
NS = 16
L = 16
CHUNK = 128
W = 128


NBUF = 4


def _bias_mf_body(latdim, b_per_w, ut_hbm, it_hbm, ub_hbm, ib_hbm, usr_hbm,
                  itm_hbm, out_hbm, usr_v, itm_v, ublk, iblk, ubv, ibv,
                  outv, sem0, sem1, sem2, sem3, bsem):
  sems = (sem0, sem1, sem2, sem3)
  wid = lax.axis_index("s") * NC + lax.axis_index("c")
  base = wid * b_per_w
  n_groups = b_per_w // L

  pltpu.sync_copy(usr_hbm.at[pl.ds(base, b_per_w)], usr_v)
  pltpu.sync_copy(itm_hbm.at[pl.ds(base, b_per_w)], itm_v)

  bias_copies = []
  for g in range(b_per_w // CHUNK):
    sl = pl.ds(g * CHUNK, CHUNK)
    bias_copies.append(
        pltpu.async_copy(ub_hbm.at[usr_v.at[sl]], ubv.at[sl], bsem))
    bias_copies.append(
        pltpu.async_copy(ib_hbm.at[itm_v.at[sl]], ibv.at[sl], bsem))

  lane = lax.iota(jnp.int32, L)
  nvec = latdim // L
  dnums = lax.GatherDimensionNumbers(
      offset_dims=(), collapsed_slice_dims=(0,), start_index_map=(0,))

  def shufxor(x, k):
    return lax.gather(x, (lane ^ k)[:, None], dnums, (1,),
                      mode=lax.GatherScatterMode.PROMISE_IN_BOUNDS)

  def fire(u_idx, i_idx, buf, sem):
    ua = pl.multiple_of(u_idx & ~(W - 1), W)
    ia = pl.multiple_of(i_idx & ~(W - 1), W)
    pltpu.async_copy(ut_hbm.at[:, pl.ds(ua, W)], ublk.at[buf], sem)
    pltpu.async_copy(it_hbm.at[:, pl.ds(ia, W)], iblk.at[buf], sem)

  def drain(buf, sem):
    pltpu.make_async_copy(
        ut_hbm.at[:, pl.ds(0, W)], ublk.at[buf], sem).wait()
    pltpu.make_async_copy(
        it_hbm.at[:, pl.ds(0, W)], iblk.at[buf], sem).wait()

  def pair_dot(buf, uoff, ioff):
    bufv = jnp.full((L,), buf, jnp.int32)
    uof = jnp.full((L,), uoff, jnp.int32)
    iof = jnp.full((L,), ioff, jnp.int32)
    acc = jnp.zeros((L,), jnp.float32)
    for c in range(nvec):
      fv = c * L + lane
      uv = plsc.load_gather(ublk, [bufv, fv, uof])
      iv = plsc.load_gather(iblk, [bufv, fv, iof])
      acc = acc + uv * iv
    for k in (1, 2, 4, 8):
      acc = acc + shufxor(acc, k)
    return acc

  for c in bias_copies:
    c.wait()

  depth = NBUF - 1
  uvec0 = usr_v[pl.ds(0, L)]
  ivec0 = itm_v[pl.ds(0, L)]
  for b in range(depth):
    fire(uvec0[b], ivec0[b], b, sems[b])

  def group(g, carry):
    uvec, ivec = carry
    sl = pl.ds(g * L, L)
    vec = ubv[sl] + ibv[sl]
    gn = jnp.minimum(g + 1, n_groups - 1)
    nsl = pl.ds(gn * L, L)
    nuvec = usr_v[nsl]
    nivec = itm_v[nsl]
    for j in range(L):
      buf = j % NBUF
      jn = j + depth
      fvec, gvec = (uvec, ivec) if jn < L else (nuvec, nivec)
      fire(fvec[jn % L], gvec[jn % L], jn % NBUF, sems[jn % NBUF])
      drain(buf, sems[buf])
      tot = pair_dot(buf, uvec[j] & (W - 1), ivec[j] & (W - 1))
      vec = jnp.where(lane == j, vec + tot, vec)
    outv[sl] = vec
    return (nuvec, nivec)

  lax.fori_loop(0, n_groups, group, (uvec0, ivec0))
  for b in range(depth):
    drain(b % NBUF, sems[b % NBUF])

  pltpu.sync_copy(outv, out_hbm.at[pl.ds(base, b_per_w)])


def kernel(uEmbeds, iEmbeds, uBias, iBias, usr, itm):
  batch = usr.shape[0]
  latdim = uEmbeds.shape[1]
  nw = NC * NS
  b_per_w = batch // nw
  uT = uEmbeds.T
  iT = iEmbeds.T
  mesh = plsc.VectorSubcoreMesh(
      core_axis_name="c", subcore_axis_name="s", num_cores=NC,
      num_subcores=NS)
  k = pl.kernel(
      functools.partial(_bias_mf_body, latdim, b_per_w),
      out_type=jax.ShapeDtypeStruct((batch,), jnp.float32),
      mesh=mesh,
      scratch_types=[
          pltpu.VMEM((b_per_w,), jnp.int32),
          pltpu.VMEM((b_per_w,), jnp.int32),
          pltpu.VMEM((NBUF, latdim, W), jnp.float32),
          pltpu.VMEM((NBUF, latdim, W), jnp.float32),
          pltpu.VMEM((b_per_w,), jnp.float32),
          pltpu.VMEM((b_per_w,), jnp.float32),
          pltpu.VMEM((b_per_w,), jnp.float32),
          pltpu.SemaphoreType.DMA,
          pltpu.SemaphoreType.DMA,
          pltpu.SemaphoreType.DMA,
          pltpu.SemaphoreType.DMA,
          pltpu.SemaphoreType.DMA,
      ],
      compiler_params=pltpu.CompilerParams(
          use_tc_tiling_on_sc=True, needs_layout_passes=False),
  )
  return k(uT, iT, uBias, iBias, usr, itm)
